# baseline (speedup 1.0000x reference)
import functools
import jax
import jax.numpy as jnp
from jax import lax
from jax.experimental import pallas as pl
from jax.experimental.pallas import tpu as pltpu

E = 160000
NC = 88000
NCYC = 16000
M1 = 176000
M2 = 88000

BR_C = 1000
BR_E = 1000


def _relu(x):
    return jnp.maximum(x, 0.0)


def _tc1_body(e2c1, e2c2, b1, b2, ca, bc,
              w20, bb20, w21, bb21, w22, bb22,
              w10, bb10, w11, bb11,
              we0, bbe0, we1, bbe1,
              eps_c,
              cycle_out, lac_out):
    x = jnp.concatenate([e2c2[...], b2[...], e2c1[...], b1[...]], axis=1)
    h = _relu(jnp.dot(x, w20[...], preferred_element_type=jnp.float32) + bb20[...])
    h = _relu(jnp.dot(h, w21[...], preferred_element_type=jnp.float32) + bb21[...])
    lift = jnp.dot(h, w22[...], preferred_element_type=jnp.float32) + bb22[...]

    s = 1.0 + eps_c[0, 0]
    cin = s * jnp.concatenate([ca[...], bc[...]], axis=1) + lift
    h = _relu(jnp.dot(cin, w10[...], preferred_element_type=jnp.float32) + bb10[...])
    cycle_out[...] = jnp.dot(h, w11[...], preferred_element_type=jnp.float32) + bb11[...]

    ein = jnp.concatenate([lift, ca[...]], axis=1)
    h = _relu(jnp.dot(ein, we0[...], preferred_element_type=jnp.float32) + bbe0[...])
    lac_out[...] = jnp.dot(h, we1[...], preferred_element_type=jnp.float32) + bbe1[...]


def _full(shape):
    return pl.BlockSpec(shape, lambda i: (0,) * len(shape))


def _rows(br, ch):
    return pl.BlockSpec((br, ch), lambda i: (i, 0))


def _tc1(e2c1, e2c2, b1, b2, ca, bc, params):
    cm2 = params["cycle_mlp_2"]
    cm1 = params["cycle_mlp_1"]
    em1 = params["edge_mlp_1"]
    wargs = [cm2[0][0], cm2[0][1], cm2[1][0], cm2[1][1], cm2[2][0], cm2[2][1],
             cm1[0][0], cm1[0][1], cm1[1][0], cm1[1][1],
             em1[0][0], em1[0][1], em1[1][0], em1[1][1],
             params["eps_cycle_1"]]
    wspecs = [_full(w.shape) for w in wargs]
    grid = NC // BR_C
    out = pl.pallas_call(
        _tc1_body,
        grid=(grid,),
        in_specs=[_rows(BR_C, 128)] * 6 + wspecs,
        out_specs=[_rows(BR_C, 128), _rows(BR_C, 128)],
        out_shape=[jax.ShapeDtypeStruct((NC, 128), jnp.float32),
                   jax.ShapeDtypeStruct((NC, 128), jnp.float32)],
    )(e2c1, e2c2, b1, b2, ca, bc, *wargs)
    return out


def _tc2_body(lac, blac, wa1, wb1, wa2, wb2, y1, y2):
    y1[...] = (jnp.dot(lac[...], wa1[...], preferred_element_type=jnp.float32)
               + jnp.dot(blac[...], wb1[...], preferred_element_type=jnp.float32))
    y2[...] = (jnp.dot(lac[...], wa2[...], preferred_element_type=jnp.float32)
               + jnp.dot(blac[...], wb2[...], preferred_element_type=jnp.float32))


def _tc2(lac, blac, params):
    w30 = params["edge_mlp_3"][0][0]
    wa1, wb1, wa2, wb2 = w30[0:128], w30[128:256], w30[256:384], w30[384:512]
    grid = NC // BR_C
    return pl.pallas_call(
        _tc2_body,
        grid=(grid,),
        in_specs=[_rows(BR_C, 128)] * 2 + [_full((128, 128))] * 4,
        out_specs=[_rows(BR_C, 128), _rows(BR_C, 128)],
        out_shape=[jax.ShapeDtypeStruct((NC, 128), jnp.float32),
                   jax.ShapeDtypeStruct((NC, 128), jnp.float32)],
    )(lac, blac, wa1, wb1, wa2, wb2)


def _tc3_body(lvl1h, edge, b30, w31, b31, w32, b32, w0, c0, w1, c1, eps_e, out):
    h = _relu(lvl1h[...] + b30[...])
    h = _relu(jnp.dot(h, w31[...], preferred_element_type=jnp.float32) + b31[...])
    la = jnp.dot(h, w32[...], preferred_element_type=jnp.float32) + b32[...]
    t = (1.0 + eps_e[0, 0]) * edge[...] + la
    h = _relu(jnp.dot(t, w0[...], preferred_element_type=jnp.float32) + c0[...])
    out[...] = jnp.dot(h, w1[...], preferred_element_type=jnp.float32) + c1[...]


def _tc3(lvl1h, edge_attr, params):
    em3 = params["edge_mlp_3"]
    em2 = params["edge_mlp_2"]
    wargs = [em3[0][1], em3[1][0], em3[1][1], em3[2][0], em3[2][1],
             em2[0][0], em2[0][1], em2[1][0], em2[1][1],
             params["eps_edge_1"]]
    wspecs = [_full(w.shape) for w in wargs]
    grid = E // BR_E
    return pl.pallas_call(
        _tc3_body,
        grid=(grid,),
        in_specs=[_rows(BR_E, 128)] * 2 + wspecs,
        out_specs=_rows(BR_E, 128),
        out_shape=jax.ShapeDtypeStruct((E, 128), jnp.float32),
    )(lvl1h, edge_attr, *wargs)


def _scatter_add(table, src, dst, nrows):
    return jax.ops.segment_sum(jnp.take(table, src, axis=0), dst,
                               num_segments=nrows)


def _gather(table, idx):
    return jnp.take(table, idx, axis=0)


def kernel(edge_attr, cycle_attr, params, cycle_ids,
           e2c_src_1, e2c_dst_1, e2c_src_2, e2c_dst_2,
           c2e_src_1, c2e_dst_1, c2e_src_2, c2e_dst_2):
    src = jnp.concatenate([e2c_src_1, e2c_src_2])
    dst = jnp.concatenate([e2c_dst_1, e2c_dst_2 + NC])
    e2c = _scatter_add(edge_attr, src, dst, 2 * NC)
    e2c1, e2c2 = e2c[:NC], e2c[NC:]

    iota = jnp.arange(NC, dtype=jnp.int32)
    seg_src = jnp.concatenate([iota, iota + NC, iota + 2 * NC])
    seg_dst = jnp.concatenate([cycle_ids, cycle_ids + NCYC, cycle_ids + 2 * NCYC])
    table3 = jnp.concatenate([e2c1, e2c2, cycle_attr], axis=0)
    segs = _scatter_add(table3, seg_src, seg_dst, 3 * NCYC)

    gidx = jnp.concatenate([cycle_ids, cycle_ids + NCYC, cycle_ids + 2 * NCYC])
    b = _gather(segs, gidx)
    b1, b2, bc = b[:NC], b[NC:2 * NC], b[2 * NC:]

    cycle_out, lac = _tc1(e2c1, e2c2, b1, b2, cycle_attr, bc, params)

    slac = _scatter_add(lac, iota, cycle_ids, NCYC)
    blac = _gather(slac, cycle_ids)

    y1, y2 = _tc2(lac, blac, params)

    ytab = jnp.concatenate([y1, y2], axis=0)
    csrc = jnp.concatenate([c2e_src_1, c2e_src_2 + NC])
    cdst = jnp.concatenate([c2e_dst_1, c2e_dst_2])
    lvl1h = _scatter_add(ytab, csrc, cdst, E)

    edge_out = _tc3(lvl1h, edge_attr, params)
    return (edge_out, cycle_out)
